# Initial kernel scaffold; baseline (speedup 1.0000x reference)
import jax
import jax.numpy as jnp
from jax.experimental import pallas as pl


def kernel(atom_types, coords_spherical, edge_index, edge_attr, emb_Wr, emb_Wi, emb_br, emb_bi, Wq_r, Wq_i, Wk_r, Wk_i, Wv_r, Wv_i, we, W1r, W1i, b1r, b1i, b_mod, W2r, W2i, b2r, b2i, out_W, out_b):
    raise NotImplementedError("write your pallas kernel here")


# speedup vs baseline: 7.2752x; 7.2752x over previous
import functools
import math

import jax
import jax.numpy as jnp
from jax import lax
from jax.experimental import pallas as pl
from jax.experimental.pallas import tpu as pltpu
from jax.experimental.pallas import tpu_sc as plsc

N = 10000
E = 320000
H = 64
FF = 128
L = 2
ED = 4

N_PAD = 10240
NW = 32
C = 128
NCHUNK = 79
CW = C * NCHUNK
E_PAD = CW * NW
ROWS_PER_SUB = N_PAD // 16

BN = 2048


def _mm_bias_kernel(x_ref, w_ref, b_ref, o_ref):
    o_ref[...] = (
        jnp.dot(x_ref[...], w_ref[...], preferred_element_type=jnp.float32)
        + b_ref[...]
    )


def _tc_matmul_bias(x, w, b):
    n, k = x.shape
    m = w.shape[1]
    return pl.pallas_call(
        _mm_bias_kernel,
        grid=(n // BN,),
        in_specs=[
            pl.BlockSpec((BN, k), lambda i: (i, 0)),
            pl.BlockSpec((k, m), lambda i: (0, 0)),
            pl.BlockSpec((1, m), lambda i: (0, 0)),
        ],
        out_specs=pl.BlockSpec((BN, m), lambda i: (i, 0)),
        out_shape=jax.ShapeDtypeStruct((n, m), jnp.float32),
    )(x, w, b)


def _qkv_kernel(z_ref, w_ref, q_ref, k_ref, v_ref):
    h = jnp.dot(z_ref[...], w_ref[...], preferred_element_type=jnp.float32)
    q_ref[...] = h[:, : 2 * H]
    k_ref[...] = h[:, 2 * H : 4 * H]
    v_ref[...] = h[:, 4 * H :]


def _tc_qkv(z, wqkv):
    return pl.pallas_call(
        _qkv_kernel,
        grid=(N_PAD // BN,),
        in_specs=[
            pl.BlockSpec((BN, 2 * H), lambda i: (i, 0)),
            pl.BlockSpec((2 * H, 6 * H), lambda i: (0, 0)),
        ],
        out_specs=[
            pl.BlockSpec((BN, 2 * H), lambda i: (i, 0)),
            pl.BlockSpec((BN, 2 * H), lambda i: (i, 0)),
            pl.BlockSpec((BN, 2 * H), lambda i: (i, 0)),
        ],
        out_shape=[jax.ShapeDtypeStruct((N_PAD, 2 * H), jnp.float32)] * 3,
    )(z, wqkv)


def _sum32_kernel(d_ref, o_ref):
    o_ref[...] = jnp.sum(d_ref[...], axis=0, keepdims=True) + 1e-9


def _tc_sum32(d32):
    bc = 2560
    return pl.pallas_call(
        _sum32_kernel,
        grid=(N_PAD // bc,),
        in_specs=[pl.BlockSpec((NW, bc), lambda i: (0, i))],
        out_specs=pl.BlockSpec((1, bc), lambda i: (0, i)),
        out_shape=jax.ShapeDtypeStruct((1, N_PAD), jnp.float32),
    )(d32)


def _ffn_kernel(z_ref, a_ref, w1_ref, b1_ref, bm_ref, w2_ref, b2_ref, o_ref):
    za = z_ref[...] + a_ref[0] + a_ref[1]
    h = jnp.dot(za, w1_ref[...], preferred_element_type=jnp.float32) + b1_ref[...]
    hr = h[:, :FF]
    hi = h[:, FF:]
    mag = jnp.sqrt(hr * hr + hi * hi + 1e-6)
    s = jnp.maximum(mag + bm_ref[...], 0.0) / mag
    hs = jnp.concatenate([hr * s, hi * s], axis=1)
    f = jnp.dot(hs, w2_ref[...], preferred_element_type=jnp.float32) + b2_ref[...]
    o_ref[...] = f + za


def _tc_ffn(z, a2, w1, b1, bm, w2, b2):
    return pl.pallas_call(
        _ffn_kernel,
        grid=(N_PAD // BN,),
        in_specs=[
            pl.BlockSpec((BN, 2 * H), lambda i: (i, 0)),
            pl.BlockSpec((2, BN, 2 * H), lambda i: (0, i, 0)),
            pl.BlockSpec((2 * H, 2 * FF), lambda i: (0, 0)),
            pl.BlockSpec((1, 2 * FF), lambda i: (0, 0)),
            pl.BlockSpec((1, FF), lambda i: (0, 0)),
            pl.BlockSpec((2 * FF, 2 * H), lambda i: (0, 0)),
            pl.BlockSpec((1, 2 * H), lambda i: (0, 0)),
        ],
        out_specs=pl.BlockSpec((BN, 2 * H), lambda i: (i, 0)),
        out_shape=jax.ShapeDtypeStruct((N_PAD, 2 * H), jnp.float32),
    )(z, a2, w1, b1, bm, w2, b2)


def _final_kernel(z_ref, wv_ref, ob_ref, o_ref, acc_ref):
    i = pl.program_id(0)

    @pl.when(i == 0)
    def _init():
        acc_ref[...] = jnp.zeros_like(acc_ref)

    z = z_ref[...]
    zr = z[:, :H]
    zi = z[:, H:]
    mz = jnp.sqrt(zr * zr + zi * zi + 1e-6)
    row = i * BN + lax.broadcasted_iota(jnp.int32, (BN, H), 0)
    mz = jnp.where(row < N, mz, 0.0)
    part = jnp.sum(mz, axis=0, keepdims=True)
    partp = jnp.concatenate([part, jnp.zeros((1, H), jnp.float32)], axis=1)
    acc_ref[0:1, :] = acc_ref[0:1, :] + partp

    o_ref[...] = jnp.zeros((8, 128), jnp.float32)

    @pl.when(i == pl.num_programs(0) - 1)
    def _fin():
        tot = jnp.sum(acc_ref[0:1, :] * wv_ref[...])
        outv = tot + float(N) * ob_ref[0, 0]
        ri = lax.broadcasted_iota(jnp.int32, (8, 128), 0)
        ci = lax.broadcasted_iota(jnp.int32, (8, 128), 1)
        o_ref[...] = jnp.where((ri == 0) & (ci == 0), outv, 0.0)


def _tc_final(z, wvec, obvec):
    return pl.pallas_call(
        _final_kernel,
        grid=(N_PAD // BN,),
        in_specs=[
            pl.BlockSpec((BN, 2 * H), lambda i: (i, 0)),
            pl.BlockSpec((1, 128), lambda i: (0, 0)),
            pl.BlockSpec((1, 128), lambda i: (0, 0)),
        ],
        out_specs=pl.BlockSpec((8, 128), lambda i: (0, 0)),
        out_shape=jax.ShapeDtypeStruct((8, 128), jnp.float32),
        scratch_shapes=[pltpu.VMEM((8, 128), jnp.float32)],
    )(z, wvec, obvec)


def _mesh():
    return plsc.VectorSubcoreMesh(
        core_axis_name="c", subcore_axis_name="s", num_cores=2, num_subcores=16
    )


_SC_PARAMS = pltpu.CompilerParams(needs_layout_passes=False)


def _worker_id():
    return lax.axis_index("s") * 2 + lax.axis_index("c")


def _pass1_body(q_hbm, k_hbm, src_hbm, dst_hbm, ea_hbm, we_hbm,
                ex_out, den_out,
                sidx, didx, qbuf, kbuf, eabuf, exbuf, den_loc, wev,
                sem_q, sem_k):
    wid = _worker_id()
    base0 = wid * CW
    zeros16 = jnp.zeros((16,), jnp.float32)

    def _zero(j, carry):
        den_loc[pl.ds(j * 16, 16)] = zeros16
        return carry

    lax.fori_loop(0, N_PAD // 16, _zero, 0)

    pltpu.sync_copy(we_hbm, wev)
    wec = [wev[pl.ds(c * 16, 16)] for c in range(ED)]

    def _chunk(t, carry):
        base = base0 + t * C
        pltpu.sync_copy(src_hbm.at[pl.ds(base, C)], sidx)
        pltpu.sync_copy(dst_hbm.at[pl.ds(base, C)], didx)
        pltpu.async_copy(q_hbm.at[didx], qbuf, sem_q).wait()
        pltpu.async_copy(k_hbm.at[sidx], kbuf, sem_k).wait()
        pltpu.sync_copy(ea_hbm.at[pl.ds(0, ED), pl.ds(base, C)], eabuf)

        lane = lax.iota(jnp.int32, 16)

        def _group(g, gcarry):
            score = zeros16
            for e in range(16):
                row = g * 16 + e
                acc = qbuf[row, pl.ds(0, 16)] * kbuf[row, pl.ds(0, 16)]
                for j in range(1, 8):
                    acc = acc + (qbuf[row, pl.ds(j * 16, 16)]
                                 * kbuf[row, pl.ds(j * 16, 16)])
                score = jnp.where(lane == e, jnp.full((16,), jnp.sum(acc)),
                                  score)
            eb = eabuf[0, pl.ds(g * 16, 16)] * wec[0]
            for cc in range(1, ED):
                eb = eb + eabuf[cc, pl.ds(g * 16, 16)] * wec[cc]
            ex16 = jnp.exp(score + eb)
            exbuf[pl.ds(g * 16, 16)] = ex16
            d16 = didx[pl.ds(g * 16, 16)]
            for e in range(16):
                d = d16[e]
                dbase = (d >> 4) << 4
                dlane = d & 15
                vec = den_loc[pl.ds(dbase, 16)]
                exv = jnp.full((16,), ex16[e])
                den_loc[pl.ds(dbase, 16)] = jnp.where(lane == dlane,
                                                      vec + exv, vec)
            return gcarry

        lax.fori_loop(0, C // 16, _group, 0)
        pltpu.sync_copy(exbuf, ex_out.at[pl.ds(base, C)])
        return carry

    lax.fori_loop(0, NCHUNK, _chunk, 0)
    pltpu.sync_copy(den_loc, den_out.at[wid])


def _sc_pass1(q, k, src, dst, ea_flat, we16):
    fn = pl.kernel(
        _pass1_body,
        out_type=[
            jax.ShapeDtypeStruct((E_PAD,), jnp.float32),
            jax.ShapeDtypeStruct((NW, N_PAD), jnp.float32),
        ],
        mesh=_mesh(),
        scratch_types=[
            pltpu.VMEM((C,), jnp.int32),
            pltpu.VMEM((C,), jnp.int32),
            pltpu.VMEM((C, 2 * H), jnp.float32),
            pltpu.VMEM((C, 2 * H), jnp.float32),
            pltpu.VMEM((ED, C), jnp.float32),
            pltpu.VMEM((C,), jnp.float32),
            pltpu.VMEM((N_PAD,), jnp.float32),
            pltpu.VMEM((ED * 16,), jnp.float32),
            pltpu.SemaphoreType.DMA,
            pltpu.SemaphoreType.DMA,
        ],
        compiler_params=_SC_PARAMS,
    )
    return fn(q, k, src, dst, ea_flat, we16)


def _pass2_body(v_hbm, ex_hbm, den_hbm, src_hbm, dst_hbm,
                a_out,
                sidx, didx, vbuf, sbuf, exbuf, denv, zbuf, a_sh,
                sem_v):
    cid = lax.axis_index("c")
    sid = lax.axis_index("s")
    wid = sid * 2 + cid
    base0 = wid * CW
    zeros16 = jnp.zeros((16,), jnp.float32)

    pltpu.sync_copy(den_hbm, denv)

    for r in range(16):
        for j in range(8):
            zbuf[r, pl.ds(j * 16, 16)] = zeros16

    def _zrow(r, carry):
        pltpu.sync_copy(zbuf, a_sh.at[pl.ds(sid * ROWS_PER_SUB + r * 16, 16)])
        return carry

    lax.fori_loop(0, ROWS_PER_SUB // 16, _zrow, 0)
    plsc.subcore_barrier()

    def _chunk(t, carry):
        base = base0 + t * C
        pltpu.sync_copy(src_hbm.at[pl.ds(base, C)], sidx)
        pltpu.sync_copy(dst_hbm.at[pl.ds(base, C)], didx)
        pltpu.async_copy(v_hbm.at[sidx], vbuf, sem_v).wait()
        pltpu.sync_copy(ex_hbm.at[pl.ds(base, C)], exbuf)

        lane = lax.iota(jnp.int32, 16)

        def _group(g, gcarry):
            d16 = didx[pl.ds(g * 16, 16)]
            den16 = jnp.full((16,), 1.0, jnp.float32)
            for e in range(16):
                d = d16[e]
                dbase = (d >> 4) << 4
                dlane = d & 15
                vec = denv[pl.ds(dbase, 16)]
                dsel = jnp.sum(jnp.where(lane == dlane, vec, zeros16))
                den16 = jnp.where(lane == e, jnp.full((16,), dsel), den16)
            ex16 = exbuf[pl.ds(g * 16, 16)]
            alpha16 = ex16 / den16
            for e in range(16):
                row = g * 16 + e
                av = jnp.full((16,), alpha16[e])
                for j in range(8):
                    sbuf[row, pl.ds(j * 16, 16)] = (
                        vbuf[row, pl.ds(j * 16, 16)] * av
                    )
            return gcarry

        lax.fori_loop(0, C // 16, _group, 0)
        pltpu.sync_copy(sbuf, a_sh.at[didx], add=True)
        return carry

    lax.fori_loop(0, NCHUNK, _chunk, 0)
    plsc.subcore_barrier()
    pltpu.sync_copy(
        a_sh.at[pl.ds(sid * ROWS_PER_SUB, ROWS_PER_SUB)],
        a_out.at[cid, pl.ds(sid * ROWS_PER_SUB, ROWS_PER_SUB)],
    )


def _sc_pass2(v, ex, den, src, dst):
    fn = pl.kernel(
        _pass2_body,
        out_type=[
            jax.ShapeDtypeStruct((2, N_PAD, 2 * H), jnp.float32),
        ],
        mesh=_mesh(),
        scratch_types=[
            pltpu.VMEM((C,), jnp.int32),
            pltpu.VMEM((C,), jnp.int32),
            pltpu.VMEM((C, 2 * H), jnp.float32),
            pltpu.VMEM((C, 2 * H), jnp.float32),
            pltpu.VMEM((C,), jnp.float32),
            pltpu.VMEM((N_PAD,), jnp.float32),
            pltpu.VMEM((16, 2 * H), jnp.float32),
            pltpu.VMEM_SHARED((N_PAD, 2 * H), jnp.float32),
            pltpu.SemaphoreType.DMA,
        ],
        compiler_params=_SC_PARAMS,
    )
    return fn(v, ex, den, src, dst)[0]


def _cplx(wr, wi):
    return jnp.concatenate(
        [jnp.concatenate([wr, wi], axis=1),
         jnp.concatenate([-wi, wr], axis=1)],
        axis=0,
    )


def kernel(atom_types, coords_spherical, edge_index, edge_attr,
           emb_Wr, emb_Wi, emb_br, emb_bi,
           Wq_r, Wq_i, Wk_r, Wk_i, Wv_r, Wv_i, we,
           W1r, W1i, b1r, b1i, b_mod, W2r, W2i, b2r, b2i,
           out_W, out_b):
    f32 = jnp.float32
    d_in = atom_types.shape[1] + 3
    k_pad = 256

    x = jnp.concatenate([atom_types, coords_spherical], axis=1)
    x_pad = jnp.zeros((N_PAD, k_pad), f32).at[:N, :d_in].set(x)
    w_emb = jnp.zeros((k_pad, 2 * H), f32).at[:d_in].set(
        jnp.concatenate([emb_Wr, emb_Wi], axis=1))
    b_emb = jnp.concatenate([emb_br, emb_bi]).reshape(1, 2 * H)

    pad_e = E_PAD - E
    pad_idx = jnp.full((pad_e,), N_PAD - 1, jnp.int32)
    src_pad = jnp.concatenate([edge_index[0].astype(jnp.int32), pad_idx])
    dst_pad = jnp.concatenate([edge_index[1].astype(jnp.int32), pad_idx])
    ea_t = jnp.concatenate(
        [edge_attr.astype(f32), jnp.zeros((pad_e, ED), f32)], axis=0
    ).T.reshape(ED, E_PAD)

    z = _tc_matmul_bias(x_pad, w_emb, b_emb)

    inv_sqrt_h = 1.0 / math.sqrt(float(H))
    for i in range(L):
        wqkv = jnp.concatenate(
            [_cplx(Wq_r[i], Wq_i[i]) * inv_sqrt_h,
             _cplx(Wk_r[i], Wk_i[i]),
             _cplx(Wv_r[i], Wv_i[i])],
            axis=1,
        )
        q, k, v = _tc_qkv(z, wqkv)
        we_exp = jnp.repeat(we[i].astype(f32), 16)

        ex, den32 = _sc_pass1(q, k, src_pad, dst_pad, ea_t, we_exp)
        den = _tc_sum32(den32).reshape(N_PAD)
        a2 = _sc_pass2(v, ex, den, src_pad, dst_pad)

        w1 = _cplx(W1r[i], W1i[i])
        b1 = jnp.concatenate([b1r[i], b1i[i]]).reshape(1, 2 * FF)
        bm = b_mod[i].reshape(1, FF)
        w2 = _cplx(W2r[i], W2i[i])
        b2 = jnp.concatenate([b2r[i], b2i[i]]).reshape(1, 2 * H)
        z = _tc_ffn(z, a2, w1, b1, bm, w2, b2)

    wvec = jnp.zeros((1, 128), f32).at[0, :H].set(out_W[:, 0])
    obvec = jnp.zeros((1, 128), f32).at[0, 0].set(out_b[0])
    out_tile = _tc_final(z, wvec, obvec)
    return out_tile[0:1, 0:1]


# speedup vs baseline: 9.7261x; 1.3369x over previous
import functools
import math

import jax
import jax.numpy as jnp
from jax import lax
from jax.experimental import pallas as pl
from jax.experimental.pallas import tpu as pltpu
from jax.experimental.pallas import tpu_sc as plsc

N = 10000
E = 320000
H = 64
FF = 128
L = 2
ED = 4

N_PAD = 10240
NW = 32
C = 128
NCHUNK = 80
PAIRS = NCHUNK // 2
CW = C * NCHUNK
E_PAD = CW * NW
ROWS_PER_SUB = N_PAD // 16

BN = 2048


def _mm_bias_kernel(x_ref, w_ref, b_ref, o_ref):
    o_ref[...] = (
        jnp.dot(x_ref[...], w_ref[...], preferred_element_type=jnp.float32)
        + b_ref[...]
    )


def _tc_matmul_bias(x, w, b):
    n, k = x.shape
    m = w.shape[1]
    return pl.pallas_call(
        _mm_bias_kernel,
        grid=(n // BN,),
        in_specs=[
            pl.BlockSpec((BN, k), lambda i: (i, 0)),
            pl.BlockSpec((k, m), lambda i: (0, 0)),
            pl.BlockSpec((1, m), lambda i: (0, 0)),
        ],
        out_specs=pl.BlockSpec((BN, m), lambda i: (i, 0)),
        out_shape=jax.ShapeDtypeStruct((n, m), jnp.float32),
    )(x, w, b)


def _qkv_kernel(z_ref, w_ref, q_ref, k_ref, v_ref):
    h = jnp.dot(z_ref[...], w_ref[...], preferred_element_type=jnp.float32)
    q_ref[...] = h[:, : 2 * H]
    k_ref[...] = h[:, 2 * H : 4 * H]
    v_ref[...] = h[:, 4 * H :]


def _tc_qkv(z, wqkv):
    return pl.pallas_call(
        _qkv_kernel,
        grid=(N_PAD // BN,),
        in_specs=[
            pl.BlockSpec((BN, 2 * H), lambda i: (i, 0)),
            pl.BlockSpec((2 * H, 6 * H), lambda i: (0, 0)),
        ],
        out_specs=[
            pl.BlockSpec((BN, 2 * H), lambda i: (i, 0)),
            pl.BlockSpec((BN, 2 * H), lambda i: (i, 0)),
            pl.BlockSpec((BN, 2 * H), lambda i: (i, 0)),
        ],
        out_shape=[jax.ShapeDtypeStruct((N_PAD, 2 * H), jnp.float32)] * 3,
    )(z, wqkv)


def _sum32_kernel(d_ref, o_ref):
    o_ref[...] = jnp.sum(d_ref[...], axis=0, keepdims=True) + 1e-9


def _tc_sum32(d32):
    bc = 2560
    return pl.pallas_call(
        _sum32_kernel,
        grid=(N_PAD // bc,),
        in_specs=[pl.BlockSpec((NW, bc), lambda i: (0, i))],
        out_specs=pl.BlockSpec((1, bc), lambda i: (0, i)),
        out_shape=jax.ShapeDtypeStruct((1, N_PAD), jnp.float32),
    )(d32)


def _ffn_kernel(z_ref, a_ref, w1_ref, b1_ref, bm_ref, w2_ref, b2_ref, o_ref):
    za = z_ref[...] + a_ref[0] + a_ref[1]
    h = jnp.dot(za, w1_ref[...], preferred_element_type=jnp.float32) + b1_ref[...]
    hr = h[:, :FF]
    hi = h[:, FF:]
    mag = jnp.sqrt(hr * hr + hi * hi + 1e-6)
    s = jnp.maximum(mag + bm_ref[...], 0.0) / mag
    hs = jnp.concatenate([hr * s, hi * s], axis=1)
    f = jnp.dot(hs, w2_ref[...], preferred_element_type=jnp.float32) + b2_ref[...]
    o_ref[...] = f + za


def _tc_ffn(z, a2, w1, b1, bm, w2, b2):
    return pl.pallas_call(
        _ffn_kernel,
        grid=(N_PAD // BN,),
        in_specs=[
            pl.BlockSpec((BN, 2 * H), lambda i: (i, 0)),
            pl.BlockSpec((2, BN, 2 * H), lambda i: (0, i, 0)),
            pl.BlockSpec((2 * H, 2 * FF), lambda i: (0, 0)),
            pl.BlockSpec((1, 2 * FF), lambda i: (0, 0)),
            pl.BlockSpec((1, FF), lambda i: (0, 0)),
            pl.BlockSpec((2 * FF, 2 * H), lambda i: (0, 0)),
            pl.BlockSpec((1, 2 * H), lambda i: (0, 0)),
        ],
        out_specs=pl.BlockSpec((BN, 2 * H), lambda i: (i, 0)),
        out_shape=jax.ShapeDtypeStruct((N_PAD, 2 * H), jnp.float32),
    )(z, a2, w1, b1, bm, w2, b2)


def _final_kernel(z_ref, wv_ref, ob_ref, o_ref, acc_ref):
    i = pl.program_id(0)

    @pl.when(i == 0)
    def _init():
        acc_ref[...] = jnp.zeros_like(acc_ref)

    z = z_ref[...]
    zr = z[:, :H]
    zi = z[:, H:]
    mz = jnp.sqrt(zr * zr + zi * zi + 1e-6)
    row = i * BN + lax.broadcasted_iota(jnp.int32, (BN, H), 0)
    mz = jnp.where(row < N, mz, 0.0)
    part = jnp.sum(mz, axis=0, keepdims=True)
    partp = jnp.concatenate([part, jnp.zeros((1, H), jnp.float32)], axis=1)
    acc_ref[0:1, :] = acc_ref[0:1, :] + partp

    o_ref[...] = jnp.zeros((8, 128), jnp.float32)

    @pl.when(i == pl.num_programs(0) - 1)
    def _fin():
        tot = jnp.sum(acc_ref[0:1, :] * wv_ref[...])
        outv = tot + float(N) * ob_ref[0, 0]
        ri = lax.broadcasted_iota(jnp.int32, (8, 128), 0)
        ci = lax.broadcasted_iota(jnp.int32, (8, 128), 1)
        o_ref[...] = jnp.where((ri == 0) & (ci == 0), outv, 0.0)


def _tc_final(z, wvec, obvec):
    return pl.pallas_call(
        _final_kernel,
        grid=(N_PAD // BN,),
        in_specs=[
            pl.BlockSpec((BN, 2 * H), lambda i: (i, 0)),
            pl.BlockSpec((1, 128), lambda i: (0, 0)),
            pl.BlockSpec((1, 128), lambda i: (0, 0)),
        ],
        out_specs=pl.BlockSpec((8, 128), lambda i: (0, 0)),
        out_shape=jax.ShapeDtypeStruct((8, 128), jnp.float32),
        scratch_shapes=[pltpu.VMEM((8, 128), jnp.float32)],
    )(z, wvec, obvec)


def _mesh():
    return plsc.VectorSubcoreMesh(
        core_axis_name="c", subcore_axis_name="s", num_cores=2, num_subcores=16
    )


_SC_PARAMS = pltpu.CompilerParams(needs_layout_passes=False)


def _worker_id():
    return lax.axis_index("s") * 2 + lax.axis_index("c")


def _pass1_body(q_hbm, k_hbm, src_hbm, dst_hbm, ea_hbm, we_hbm,
                ex_out, den_out,
                sidx0, sidx1, didx0, didx1, didxc,
                qbuf0, qbuf1, kbuf0, kbuf1, eabuf0, eabuf1, exbuf0, exbuf1,
                den_loc, wev,
                s_si0, s_si1, s_di0, s_di1, s_q0, s_q1, s_k0, s_k1,
                s_ea0, s_ea1, s_xw0, s_xw1):
    sidx = [sidx0, sidx1]
    didx = [didx0, didx1]
    qbuf = [qbuf0, qbuf1]
    kbuf = [kbuf0, kbuf1]
    eabuf = [eabuf0, eabuf1]
    exbuf = [exbuf0, exbuf1]
    s_si = [s_si0, s_si1]
    s_di = [s_di0, s_di1]
    s_q = [s_q0, s_q1]
    s_k = [s_k0, s_k1]
    s_ea = [s_ea0, s_ea1]
    s_xw = [s_xw0, s_xw1]

    wid = _worker_id()
    base0 = wid * CW
    lane = lax.iota(jnp.int32, 16)
    zeros16 = jnp.zeros((16,), jnp.float32)

    def _zero(j, carry):
        den_loc[pl.ds(j * 16, 16)] = zeros16
        return carry

    lax.fori_loop(0, N_PAD // 16, _zero, 0)

    pltpu.sync_copy(we_hbm, wev)
    wec = [wev[pl.ds(c * 16, 16)] for c in range(ED)]

    def issue_idx(t, b):
        base = base0 + t * C
        pltpu.async_copy(src_hbm.at[pl.ds(base, C)], sidx[b], s_si[b])
        pltpu.async_copy(dst_hbm.at[pl.ds(base, C)], didx[b], s_di[b])

    def wait_idx(b):
        pltpu.make_async_copy(src_hbm.at[pl.ds(0, C)], sidx[b], s_si[b]).wait()
        pltpu.make_async_copy(dst_hbm.at[pl.ds(0, C)], didx[b], s_di[b]).wait()

    def issue_gather(t, b):
        base = base0 + t * C
        pltpu.async_copy(q_hbm.at[didx[b]], qbuf[b], s_q[b])
        pltpu.async_copy(k_hbm.at[sidx[b]], kbuf[b], s_k[b])
        pltpu.async_copy(ea_hbm.at[pl.ds(0, ED), pl.ds(base, C)],
                         eabuf[b], s_ea[b])

    def wait_gather(b):
        pltpu.make_async_copy(q_hbm.at[didx[b]], qbuf[b], s_q[b]).wait()
        pltpu.make_async_copy(k_hbm.at[sidx[b]], kbuf[b], s_k[b]).wait()
        pltpu.make_async_copy(ea_hbm.at[pl.ds(0, ED), pl.ds(0, C)],
                              eabuf[b], s_ea[b]).wait()

    def compute(t, b):
        for gg in range(C // 16):
            didxc[pl.ds(gg * 16, 16)] = didx[b][pl.ds(gg * 16, 16)]

        @pl.when(t < NCHUNK - 2)
        def _pref():
            issue_idx(t + 2, b)

        @pl.when(t >= 2)
        def _wb():
            pltpu.make_async_copy(exbuf[b], ex_out.at[pl.ds(0, C)],
                                  s_xw[b]).wait()

        qb = qbuf[b]
        kb = kbuf[b]
        eab = eabuf[b]
        exb = exbuf[b]

        def _group(g, gcarry):
            score = zeros16
            for e in range(16):
                row = g * 16 + e
                acc = qb[row, pl.ds(0, 16)] * kb[row, pl.ds(0, 16)]
                for j in range(1, 8):
                    acc = acc + (qb[row, pl.ds(j * 16, 16)]
                                 * kb[row, pl.ds(j * 16, 16)])
                score = jnp.where(lane == e, jnp.full((16,), jnp.sum(acc)),
                                  score)
            eb = eab[0, pl.ds(g * 16, 16)] * wec[0]
            for cc in range(1, ED):
                eb = eb + eab[cc, pl.ds(g * 16, 16)] * wec[cc]
            ex16 = jnp.exp(score + eb)
            exb[pl.ds(g * 16, 16)] = ex16
            d16 = didxc[pl.ds(g * 16, 16)]
            for e in range(16):
                d = d16[e]
                dbase = (d >> 4) << 4
                dlane = d & 15
                vec = den_loc[pl.ds(dbase, 16)]
                exv = jnp.full((16,), ex16[e])
                den_loc[pl.ds(dbase, 16)] = jnp.where(lane == dlane,
                                                      vec + exv, vec)
            return gcarry

        lax.fori_loop(0, C // 16, _group, 0)
        base = base0 + t * C
        pltpu.async_copy(exb, ex_out.at[pl.ds(base, C)], s_xw[b])

    issue_idx(0, 0)
    issue_idx(1, 1)
    wait_idx(0)
    issue_gather(0, 0)

    def _pair(p, carry):
        t0 = 2 * p
        wait_idx(1)
        issue_gather(t0 + 1, 1)
        wait_gather(0)
        compute(t0, 0)

        @pl.when(p < PAIRS - 1)
        def _nxt():
            wait_idx(0)
            issue_gather(t0 + 2, 0)

        wait_gather(1)
        compute(t0 + 1, 1)
        return carry

    lax.fori_loop(0, PAIRS, _pair, 0)
    pltpu.make_async_copy(exbuf[0], ex_out.at[pl.ds(0, C)], s_xw[0]).wait()
    pltpu.make_async_copy(exbuf[1], ex_out.at[pl.ds(0, C)], s_xw[1]).wait()
    pltpu.sync_copy(den_loc, den_out.at[wid])


def _sc_pass1(q, k, src, dst, ea_flat, we16):
    fn = pl.kernel(
        _pass1_body,
        out_type=[
            jax.ShapeDtypeStruct((E_PAD,), jnp.float32),
            jax.ShapeDtypeStruct((NW, N_PAD), jnp.float32),
        ],
        mesh=_mesh(),
        scratch_types=[
            pltpu.VMEM((C,), jnp.int32),
            pltpu.VMEM((C,), jnp.int32),
            pltpu.VMEM((C,), jnp.int32),
            pltpu.VMEM((C,), jnp.int32),
            pltpu.VMEM((C,), jnp.int32),
            pltpu.VMEM((C, 2 * H), jnp.float32),
            pltpu.VMEM((C, 2 * H), jnp.float32),
            pltpu.VMEM((C, 2 * H), jnp.float32),
            pltpu.VMEM((C, 2 * H), jnp.float32),
            pltpu.VMEM((ED, C), jnp.float32),
            pltpu.VMEM((ED, C), jnp.float32),
            pltpu.VMEM((C,), jnp.float32),
            pltpu.VMEM((C,), jnp.float32),
            pltpu.VMEM((N_PAD,), jnp.float32),
            pltpu.VMEM((ED * 16,), jnp.float32),
        ] + [pltpu.SemaphoreType.DMA] * 12,
        compiler_params=_SC_PARAMS,
    )
    return fn(q, k, src, dst, ea_flat, we16)


def _pass2_body(v_hbm, ex_hbm, den_hbm, src_hbm, dst_hbm,
                a_out,
                sidx0, sidx1, didx0, didx1, sdidx0, sdidx1,
                vbuf0, vbuf1, exbuf0, exbuf1,
                denv, zbuf, a_sh,
                s_si0, s_si1, s_di0, s_di1, s_v0, s_v1, s_x0, s_x1,
                s_sc0, s_sc1):
    sidx = [sidx0, sidx1]
    didx = [didx0, didx1]
    sdidx = [sdidx0, sdidx1]
    vbuf = [vbuf0, vbuf1]
    exbuf = [exbuf0, exbuf1]
    s_si = [s_si0, s_si1]
    s_di = [s_di0, s_di1]
    s_v = [s_v0, s_v1]
    s_x = [s_x0, s_x1]
    s_sc = [s_sc0, s_sc1]

    cid = lax.axis_index("c")
    sid = lax.axis_index("s")
    wid = sid * 2 + cid
    base0 = wid * CW
    lane = lax.iota(jnp.int32, 16)
    zeros16 = jnp.zeros((16,), jnp.float32)

    pltpu.sync_copy(den_hbm, denv)

    for r in range(16):
        for j in range(8):
            zbuf[r, pl.ds(j * 16, 16)] = zeros16

    def _zrow(r, carry):
        pltpu.sync_copy(zbuf, a_sh.at[pl.ds(sid * ROWS_PER_SUB + r * 16, 16)])
        return carry

    lax.fori_loop(0, ROWS_PER_SUB // 16, _zrow, 0)
    plsc.subcore_barrier()

    def issue_idx(t, b):
        base = base0 + t * C
        pltpu.async_copy(src_hbm.at[pl.ds(base, C)], sidx[b], s_si[b])
        pltpu.async_copy(dst_hbm.at[pl.ds(base, C)], didx[b], s_di[b])

    def wait_idx(b):
        pltpu.make_async_copy(src_hbm.at[pl.ds(0, C)], sidx[b], s_si[b]).wait()
        pltpu.make_async_copy(dst_hbm.at[pl.ds(0, C)], didx[b], s_di[b]).wait()

    def issue_gather(t, b):
        base = base0 + t * C
        pltpu.async_copy(v_hbm.at[sidx[b]], vbuf[b], s_v[b])
        pltpu.async_copy(ex_hbm.at[pl.ds(base, C)], exbuf[b], s_x[b])

    def wait_gather(b):
        pltpu.make_async_copy(v_hbm.at[sidx[b]], vbuf[b], s_v[b]).wait()
        pltpu.make_async_copy(ex_hbm.at[pl.ds(0, C)], exbuf[b], s_x[b]).wait()

    def issue_scatter(b):
        pltpu.async_copy(vbuf[b], a_sh.at[sdidx[b]], s_sc[b], add=True)

    def wait_scatter(b):
        pltpu.make_async_copy(vbuf[b], a_sh.at[sdidx[b]], s_sc[b]).wait()

    def compute(t, b):
        for gg in range(C // 16):
            sdidx[b][pl.ds(gg * 16, 16)] = didx[b][pl.ds(gg * 16, 16)]

        @pl.when(t < NCHUNK - 2)
        def _pref():
            issue_idx(t + 2, b)

        vb = vbuf[b]
        exb = exbuf[b]
        sdb = sdidx[b]

        def _group(g, gcarry):
            d16 = sdb[pl.ds(g * 16, 16)]
            den16 = jnp.full((16,), 1.0, jnp.float32)
            for e in range(16):
                d = d16[e]
                dbase = (d >> 4) << 4
                dlane = d & 15
                vec = denv[pl.ds(dbase, 16)]
                dsel = jnp.sum(jnp.where(lane == dlane, vec, zeros16))
                den16 = jnp.where(lane == e, jnp.full((16,), dsel), den16)
            ex16 = exb[pl.ds(g * 16, 16)]
            alpha16 = ex16 / den16
            for e in range(16):
                row = g * 16 + e
                av = jnp.full((16,), alpha16[e])
                for j in range(8):
                    vb[row, pl.ds(j * 16, 16)] = (
                        vb[row, pl.ds(j * 16, 16)] * av
                    )
            return gcarry

        lax.fori_loop(0, C // 16, _group, 0)

    issue_idx(0, 0)
    issue_idx(1, 1)
    wait_idx(0)
    issue_gather(0, 0)

    def _pair(p, carry):
        t0 = 2 * p

        @pl.when(p > 0)
        def _ws1():
            wait_scatter(1)

        wait_idx(1)
        issue_gather(t0 + 1, 1)
        wait_gather(0)
        compute(t0, 0)
        issue_scatter(0)

        @pl.when(p < PAIRS - 1)
        def _nxt():
            wait_scatter(0)
            wait_idx(0)
            issue_gather(t0 + 2, 0)

        wait_gather(1)
        compute(t0 + 1, 1)
        issue_scatter(1)
        return carry

    lax.fori_loop(0, PAIRS, _pair, 0)
    wait_scatter(0)
    wait_scatter(1)
    plsc.subcore_barrier()
    pltpu.sync_copy(
        a_sh.at[pl.ds(sid * ROWS_PER_SUB, ROWS_PER_SUB)],
        a_out.at[cid, pl.ds(sid * ROWS_PER_SUB, ROWS_PER_SUB)],
    )


def _sc_pass2(v, ex, den, src, dst):
    fn = pl.kernel(
        _pass2_body,
        out_type=[
            jax.ShapeDtypeStruct((2, N_PAD, 2 * H), jnp.float32),
        ],
        mesh=_mesh(),
        scratch_types=[
            pltpu.VMEM((C,), jnp.int32),
            pltpu.VMEM((C,), jnp.int32),
            pltpu.VMEM((C,), jnp.int32),
            pltpu.VMEM((C,), jnp.int32),
            pltpu.VMEM((C,), jnp.int32),
            pltpu.VMEM((C,), jnp.int32),
            pltpu.VMEM((C, 2 * H), jnp.float32),
            pltpu.VMEM((C, 2 * H), jnp.float32),
            pltpu.VMEM((C,), jnp.float32),
            pltpu.VMEM((C,), jnp.float32),
            pltpu.VMEM((N_PAD,), jnp.float32),
            pltpu.VMEM((16, 2 * H), jnp.float32),
            pltpu.VMEM_SHARED((N_PAD, 2 * H), jnp.float32),
        ] + [pltpu.SemaphoreType.DMA] * 10,
        compiler_params=_SC_PARAMS,
    )
    return fn(v, ex, den, src, dst)[0]


def _cplx(wr, wi):
    return jnp.concatenate(
        [jnp.concatenate([wr, wi], axis=1),
         jnp.concatenate([-wi, wr], axis=1)],
        axis=0,
    )


def kernel(atom_types, coords_spherical, edge_index, edge_attr,
           emb_Wr, emb_Wi, emb_br, emb_bi,
           Wq_r, Wq_i, Wk_r, Wk_i, Wv_r, Wv_i, we,
           W1r, W1i, b1r, b1i, b_mod, W2r, W2i, b2r, b2i,
           out_W, out_b):
    f32 = jnp.float32
    d_in = atom_types.shape[1] + 3
    k_pad = 256

    x = jnp.concatenate([atom_types, coords_spherical], axis=1)
    x_pad = jnp.zeros((N_PAD, k_pad), f32).at[:N, :d_in].set(x)
    w_emb = jnp.zeros((k_pad, 2 * H), f32).at[:d_in].set(
        jnp.concatenate([emb_Wr, emb_Wi], axis=1))
    b_emb = jnp.concatenate([emb_br, emb_bi]).reshape(1, 2 * H)

    pad_e = E_PAD - E
    pad_idx = jnp.full((pad_e,), N_PAD - 1, jnp.int32)
    src_pad = jnp.concatenate([edge_index[0].astype(jnp.int32), pad_idx])
    dst_pad = jnp.concatenate([edge_index[1].astype(jnp.int32), pad_idx])
    ea_t = jnp.concatenate(
        [edge_attr.astype(f32), jnp.zeros((pad_e, ED), f32)], axis=0
    ).T.reshape(ED, E_PAD)

    z = _tc_matmul_bias(x_pad, w_emb, b_emb)

    inv_sqrt_h = 1.0 / math.sqrt(float(H))
    for i in range(L):
        wqkv = jnp.concatenate(
            [_cplx(Wq_r[i], Wq_i[i]) * inv_sqrt_h,
             _cplx(Wk_r[i], Wk_i[i]),
             _cplx(Wv_r[i], Wv_i[i])],
            axis=1,
        )
        q, k, v = _tc_qkv(z, wqkv)
        we_exp = jnp.repeat(we[i].astype(f32), 16)

        ex, den32 = _sc_pass1(q, k, src_pad, dst_pad, ea_t, we_exp)
        den = _tc_sum32(den32).reshape(N_PAD)
        a2 = _sc_pass2(v, ex, den, src_pad, dst_pad)

        w1 = _cplx(W1r[i], W1i[i])
        b1 = jnp.concatenate([b1r[i], b1i[i]]).reshape(1, 2 * FF)
        bm = b_mod[i].reshape(1, FF)
        w2 = _cplx(W2r[i], W2i[i])
        b2 = jnp.concatenate([b2r[i], b2i[i]]).reshape(1, 2 * H)
        z = _tc_ffn(z, a2, w1, b1, bm, w2, b2)

    wvec = jnp.zeros((1, 128), f32).at[0, :H].set(out_W[:, 0])
    obvec = jnp.zeros((1, 128), f32).at[0, 0].set(out_b[0])
    out_tile = _tc_final(z, wvec, obvec)
    return out_tile[0:1, 0:1]
